# baseline (speedup 1.0000x reference)
import functools

import jax
import jax.numpy as jnp
from jax.experimental import pallas as pl
from jax.experimental.pallas import tpu as pltpu

_BQ = 256


def _cope_body(q_ref, a_ref, pe_ref, o_ref, *, skv: int, npos: int):
    t = jnp.dot(q_ref[...], pe_ref[...], preferred_element_type=jnp.float32)
    d = jnp.concatenate(
        [t[:, 1:] - t[:, :-1], jnp.zeros((t.shape[0], 1), jnp.float32)], axis=1
    )
    table = jnp.concatenate([t, d], axis=1)

    g = jax.nn.sigmoid(a_ref[...])
    bq = g.shape[0]
    gp = jnp.concatenate([g, jnp.zeros((bq, 128), jnp.float32)], axis=1)
    sh = 1
    while sh < 128:
        gp = gp + pltpu.roll(gp, gp.shape[1] - sh, axis=1)
        sh *= 2
    ncol = skv // 128
    cols = [gp[:, c * 128 : (c + 1) * 128] for c in range(ncol)]
    acc = cols[ncol - 1]
    out_cols = [None] * ncol
    out_cols[ncol - 1] = acc
    for c in range(ncol - 2, -1, -1):
        acc = cols[c] + acc
        out_cols[c] = acc
    s = jnp.concatenate(out_cols, axis=1)
    pos = jnp.minimum(s, float(npos - 1))
    pf = jnp.floor(pos)
    idx = pf.astype(jnp.int32)
    frac = pos - pf
    tv = jnp.take_along_axis(table, idx, axis=1)
    dv = jnp.take_along_axis(table, idx + npos, axis=1)
    o_ref[...] = tv + frac * dv


def kernel(query, attn_logits, pos_emb):
    b, h, sq, dim = query.shape
    skv = attn_logits.shape[-1]
    npos = pos_emb.shape[-1]
    rows = b * h * sq
    q2 = query.reshape(rows, dim)
    a2 = attn_logits.reshape(rows, skv)
    pe = pos_emb.reshape(dim, npos)

    body = functools.partial(_cope_body, skv=skv, npos=npos)
    out = pl.pallas_call(
        body,
        grid=(rows // _BQ,),
        in_specs=[
            pl.BlockSpec((_BQ, dim), lambda i: (i, 0)),
            pl.BlockSpec((_BQ, skv), lambda i: (i, 0)),
            pl.BlockSpec((dim, npos), lambda i: (0, 0)),
        ],
        out_specs=pl.BlockSpec((_BQ, skv), lambda i: (i, 0)),
        out_shape=jax.ShapeDtypeStruct((rows, skv), jnp.float32),
    )(q2, a2, pe)
    return out.reshape(b, h, sq, skv)


# speedup vs baseline: 1.7735x; 1.7735x over previous
import functools

import jax
import jax.numpy as jnp
import numpy as np
from jax.experimental import pallas as pl

_BQ = 256
_C = 128


def _suffix_weights() -> np.ndarray:
    i = np.arange(_C)
    ut = (i[:, None] >= i[None, :]).astype(np.float32)
    r = np.zeros((2 * _C, 2 * _C), np.float32)
    r[:_C, :_C] = ut
    r[_C:, :_C] = 1.0
    r[_C:, _C:] = ut
    return r


def _cope_body(q_ref, a_ref, pe_ref, w_ref, o_ref, *, skv: int, npos: int):
    t = jnp.dot(q_ref[...], pe_ref[...], preferred_element_type=jnp.float32)
    d = jnp.concatenate(
        [t[:, 1:] - t[:, :-1], jnp.zeros((t.shape[0], 1), jnp.float32)], axis=1
    )
    table = jnp.concatenate([t, d], axis=1)

    g = jax.nn.sigmoid(a_ref[...])
    g_hi = g.astype(jnp.bfloat16)
    g_lo = (g - g_hi.astype(jnp.float32)).astype(jnp.bfloat16)
    w = w_ref[...]

    npair = skv // (2 * _C)
    pairs = []
    for p in range(npair):
        lo, hi = p * 2 * _C, (p + 1) * 2 * _C
        acc = jnp.dot(g_hi[:, lo:hi], w, preferred_element_type=jnp.float32)
        acc = acc + jnp.dot(g_lo[:, lo:hi], w, preferred_element_type=jnp.float32)
        pairs.append(acc)

    carry = jnp.zeros((pairs[0].shape[0], 1), jnp.float32)
    s_cols = [None] * npair
    for p in range(npair - 1, -1, -1):
        s_cols[p] = pairs[p] + carry
        carry = carry + pairs[p][:, 0:1]
    s = jnp.concatenate(s_cols, axis=1)

    pos = jnp.minimum(s, float(npos - 1))
    idx = pos.astype(jnp.int32)
    frac = pos - idx.astype(jnp.float32)
    tv = jnp.take_along_axis(table, idx, axis=1)
    dv = jnp.take_along_axis(table, idx + npos, axis=1)
    o_ref[...] = tv + frac * dv


def kernel(query, attn_logits, pos_emb):
    b, h, sq, dim = query.shape
    skv = attn_logits.shape[-1]
    npos = pos_emb.shape[-1]
    rows = b * h * sq
    q2 = query.reshape(rows, dim)
    a2 = attn_logits.reshape(rows, skv)
    pe = pos_emb.reshape(dim, npos)
    w = jnp.asarray(_suffix_weights(), dtype=jnp.bfloat16)

    body = functools.partial(_cope_body, skv=skv, npos=npos)
    out = pl.pallas_call(
        body,
        grid=(rows // _BQ,),
        in_specs=[
            pl.BlockSpec((_BQ, dim), lambda i: (i, 0)),
            pl.BlockSpec((_BQ, skv), lambda i: (i, 0)),
            pl.BlockSpec((dim, npos), lambda i: (0, 0)),
            pl.BlockSpec((2 * _C, 2 * _C), lambda i: (0, 0)),
        ],
        out_specs=pl.BlockSpec((_BQ, skv), lambda i: (i, 0)),
        out_shape=jax.ShapeDtypeStruct((rows, skv), jnp.float32),
    )(q2, a2, pe, w)
    return out.reshape(b, h, sq, skv)


# speedup vs baseline: 2.7238x; 1.5358x over previous
import functools

import jax
import jax.numpy as jnp
import numpy as np
from jax.experimental import pallas as pl
from jax.experimental.pallas import tpu as pltpu

_BQ = 256
_C = 128


def _suffix_weights() -> np.ndarray:
    i = np.arange(_C)
    ut = (i[:, None] >= i[None, :]).astype(np.float32)
    r = np.zeros((2 * _C, 2 * _C), np.float32)
    r[:_C, :_C] = ut
    r[_C:, :_C] = 1.0
    r[_C:, _C:] = ut
    return r


def _cope_body(q_ref, a_ref, pe_ref, w_ref, o_ref, *, skv: int, npos: int):
    t = jnp.dot(q_ref[...], pe_ref[...], preferred_element_type=jnp.float32)
    d = pltpu.roll(t, npos - 1, axis=1) - t
    tw = jax.lax.bitcast_convert_type(
        t.astype(jnp.bfloat16), jnp.uint16
    ).astype(jnp.uint32)
    dw = jax.lax.bitcast_convert_type(
        d.astype(jnp.bfloat16), jnp.uint16
    ).astype(jnp.uint32)
    packed = (tw << 16) | dw
    packed2 = jnp.concatenate([packed, packed], axis=1)

    g = jax.nn.sigmoid(a_ref[...])
    g_hi = g.astype(jnp.bfloat16)
    g_lo = (g - g_hi.astype(jnp.float32)).astype(jnp.bfloat16)
    w = w_ref[...]

    npair = skv // (2 * _C)
    pairs = []
    for p in range(npair):
        lo, hi = p * 2 * _C, (p + 1) * 2 * _C
        acc = jnp.dot(g_hi[:, lo:hi], w, preferred_element_type=jnp.float32)
        acc = acc + jnp.dot(g_lo[:, lo:hi], w, preferred_element_type=jnp.float32)
        pairs.append(acc)

    carry = jnp.zeros((pairs[0].shape[0], 1), jnp.float32)
    s_list = [None] * npair
    for p in range(npair - 1, -1, -1):
        s_list[p] = pairs[p] + carry
        carry = carry + pairs[p][:, 0:1]
    for p in range(npair):
        pos = jnp.minimum(s_list[p], float(npos - 1))
        idx = pos.astype(jnp.int32)
        frac = pos - idx.astype(jnp.float32)
        w2 = jnp.take_along_axis(packed2, idx, axis=1, mode="promise_in_bounds")
        tv = jax.lax.bitcast_convert_type(w2 & jnp.uint32(0xFFFF0000), jnp.float32)
        dv = jax.lax.bitcast_convert_type(w2 << 16, jnp.float32)
        o_ref[:, p * 2 * _C : (p + 1) * 2 * _C] = tv + frac * dv


def kernel(query, attn_logits, pos_emb):
    b, h, sq, dim = query.shape
    skv = attn_logits.shape[-1]
    npos = pos_emb.shape[-1]
    rows = b * h * sq
    q2 = query.reshape(rows, dim)
    a2 = attn_logits.reshape(rows, skv)
    pe = pos_emb.reshape(dim, npos)
    w = jnp.asarray(_suffix_weights(), dtype=jnp.bfloat16)

    body = functools.partial(_cope_body, skv=skv, npos=npos)
    out = pl.pallas_call(
        body,
        grid=(rows // _BQ,),
        in_specs=[
            pl.BlockSpec((_BQ, dim), lambda i: (i, 0)),
            pl.BlockSpec((_BQ, skv), lambda i: (i, 0)),
            pl.BlockSpec((dim, npos), lambda i: (0, 0)),
            pl.BlockSpec((2 * _C, 2 * _C), lambda i: (0, 0)),
        ],
        out_specs=pl.BlockSpec((_BQ, skv), lambda i: (i, 0)),
        out_shape=jax.ShapeDtypeStruct((rows, skv), jnp.float32),
    )(q2, a2, pe, w)
    return out.reshape(b, h, sq, skv)


# speedup vs baseline: 2.7318x; 1.0029x over previous
import functools

import jax
import jax.numpy as jnp
import numpy as np
from jax.experimental import pallas as pl
from jax.experimental.pallas import tpu as pltpu

_BQ = 512
_C = 128


def _suffix_weights() -> np.ndarray:
    i = np.arange(_C)
    ut = (i[:, None] >= i[None, :]).astype(np.float32)
    r = np.zeros((2 * _C, 2 * _C), np.float32)
    r[:_C, :_C] = ut
    r[_C:, :_C] = 1.0
    r[_C:, _C:] = ut
    return r


def _cope_body(q_ref, a_ref, pe_ref, w_ref, o_ref, *, skv: int, npos: int):
    t = jnp.dot(q_ref[...], pe_ref[...], preferred_element_type=jnp.float32)
    d = pltpu.roll(t, npos - 1, axis=1) - t
    tw = jax.lax.bitcast_convert_type(
        t.astype(jnp.bfloat16), jnp.uint16
    ).astype(jnp.uint32)
    dw = jax.lax.bitcast_convert_type(
        d.astype(jnp.bfloat16), jnp.uint16
    ).astype(jnp.uint32)
    packed = (tw << 16) | dw
    packed2 = jnp.concatenate([packed, packed], axis=1)

    g = jax.nn.sigmoid(a_ref[...])
    g_hi = g.astype(jnp.bfloat16)
    g_lo = (g - g_hi.astype(jnp.float32)).astype(jnp.bfloat16)
    w = w_ref[...]

    npair = skv // (2 * _C)
    pairs = []
    for p in range(npair):
        lo, hi = p * 2 * _C, (p + 1) * 2 * _C
        acc = jnp.dot(g_hi[:, lo:hi], w, preferred_element_type=jnp.float32)
        acc = acc + jnp.dot(g_lo[:, lo:hi], w, preferred_element_type=jnp.float32)
        pairs.append(acc)

    carry = jnp.zeros((pairs[0].shape[0], 1), jnp.float32)
    s_list = [None] * npair
    for p in range(npair - 1, -1, -1):
        s_list[p] = pairs[p] + carry
        carry = carry + pairs[p][:, 0:1]
    for p in range(npair):
        pos = jnp.minimum(s_list[p], float(npos - 1))
        idx = pos.astype(jnp.int32)
        frac = pos - idx.astype(jnp.float32)
        w2 = jnp.take_along_axis(packed2, idx, axis=1, mode="promise_in_bounds")
        tv = jax.lax.bitcast_convert_type(w2 & jnp.uint32(0xFFFF0000), jnp.float32)
        dv = jax.lax.bitcast_convert_type(w2 << 16, jnp.float32)
        o_ref[:, p * 2 * _C : (p + 1) * 2 * _C] = tv + frac * dv


def kernel(query, attn_logits, pos_emb):
    b, h, sq, dim = query.shape
    skv = attn_logits.shape[-1]
    npos = pos_emb.shape[-1]
    rows = b * h * sq
    q2 = query.reshape(rows, dim)
    a2 = attn_logits.reshape(rows, skv)
    pe = pos_emb.reshape(dim, npos)
    w = jnp.asarray(_suffix_weights(), dtype=jnp.bfloat16)

    body = functools.partial(_cope_body, skv=skv, npos=npos)
    out = pl.pallas_call(
        body,
        grid=(rows // _BQ,),
        in_specs=[
            pl.BlockSpec((_BQ, dim), lambda i: (i, 0)),
            pl.BlockSpec((_BQ, skv), lambda i: (i, 0)),
            pl.BlockSpec((dim, npos), lambda i: (0, 0)),
            pl.BlockSpec((2 * _C, 2 * _C), lambda i: (0, 0)),
        ],
        out_specs=pl.BlockSpec((_BQ, skv), lambda i: (i, 0)),
        out_shape=jax.ShapeDtypeStruct((rows, skv), jnp.float32),
    )(q2, a2, pe, w)
    return out.reshape(b, h, sq, skv)
